# baseline (speedup 1.0000x reference)
import functools

import jax
import jax.numpy as jnp
from jax import lax
from jax.experimental import pallas as pl
from jax.experimental.pallas import tpu as pltpu
from jax.experimental.pallas import tpu_sc as plsc

NC = 2
NS = 16
NW = NC * NS
LANES = 16

B = 2
NVOX = 128 * 128 * 128
CH = 7
VPW = NVOX // NW
VC = 4096
NCHUNK = VPW // VC
TOT = B * NCHUNK
GROUPS = VC // LANES

_MAX_CH = (0, 1, 6)
_MEAN_CH = (2, 3, 4, 5)
P_ROWS = 3 * B * CH
P_FLAT = P_ROWS * LANES


def _sc_body(pred_hbm, mask_hbm, out_hbm, mb0, mb1, pb0, pb1, obuf,
             sm0, sm1, sp0, sp1):
  cid = lax.axis_index("c")
  sid = lax.axis_index("s")
  w = sid * NC + cid
  vw0 = w * VPW

  mbufs = (mb0, mb1)
  pbufs = (pb0, pb1)
  msems = (sm0, sm1)
  psems = (sp0, sp1)

  neg_inf = jnp.float32(-jnp.inf)
  zero = jnp.zeros((LANES,), jnp.float32)
  one = jnp.float32(1.0)
  idx7 = lax.iota(jnp.int32, LANES) * 7

  def start(step):
    b, k = divmod(step, NCHUNK)
    par = step % 2
    v0 = b * NVOX + vw0 + k * VC
    cm = pltpu.async_copy(mask_hbm.at[pl.ds(v0 * CH, VC * CH)],
                          mbufs[par], msems[par])
    cp = pltpu.async_copy(pred_hbm.at[pl.ds(v0, VC)],
                          pbufs[par], psems[par])
    return cm, cp

  pending = start(0)
  cnts = sums = maxs = None
  for step in range(TOT):
    b, k = divmod(step, NCHUNK)
    par = step % 2
    if k == 0:
      cnts = [zero] * CH
      sums = {ch: zero for ch in _MEAN_CH}
      maxs = {ch: jnp.full((LANES,), neg_inf) for ch in _MAX_CH}
    nxt = start(step + 1) if step + 1 < TOT else None
    pending[0].wait()
    pending[1].wait()
    pending = nxt
    mb = mbufs[par]
    pb = pbufs[par]

    def inner(g, carry, mb=mb, pb=pb):
      cnts = list(carry[:CH])
      sums = dict(zip(_MEAN_CH, carry[CH:CH + 4]))
      maxs = dict(zip(_MAX_CH, carry[CH + 4:]))
      pv = pb[pl.ds(g * LANES, LANES)]
      base = g * (LANES * CH)
      for ch in range(CH):
        mv = plsc.load_gather(mb, [idx7 + (base + ch)])
        m = mv > 0.0
        cnts[ch] = cnts[ch] + jnp.where(m, one, 0.0)
        if ch in _MEAN_CH:
          sums[ch] = sums[ch] + jnp.where(m, pv, 0.0)
        else:
          maxs[ch] = jnp.maximum(maxs[ch], jnp.where(m, pv, neg_inf))
      return tuple(cnts) + tuple(sums[c] for c in _MEAN_CH) + tuple(
          maxs[c] for c in _MAX_CH)

    carry = tuple(cnts) + tuple(sums[c] for c in _MEAN_CH) + tuple(
        maxs[c] for c in _MAX_CH)
    carry = lax.fori_loop(0, GROUPS, inner, carry)
    cnts = list(carry[:CH])
    sums = dict(zip(_MEAN_CH, carry[CH:CH + 4]))
    maxs = dict(zip(_MAX_CH, carry[CH + 4:]))

    if k == NCHUNK - 1:
      for ch in range(CH):
        r0 = (0 * B * CH + b * CH + ch) * LANES
        r1 = (1 * B * CH + b * CH + ch) * LANES
        r2 = (2 * B * CH + b * CH + ch) * LANES
        obuf[pl.ds(r0, LANES)] = sums[ch] if ch in _MEAN_CH else zero
        obuf[pl.ds(r1, LANES)] = cnts[ch]
        obuf[pl.ds(r2, LANES)] = (
            maxs[ch] if ch in _MAX_CH else jnp.full((LANES,), neg_inf))

  pltpu.sync_copy(obuf, out_hbm.at[pl.ds(w * P_FLAT, P_FLAT)])


_sc_partials = functools.partial(
    pl.kernel,
    out_type=jax.ShapeDtypeStruct((NW * P_FLAT,), jnp.float32),
    mesh=plsc.VectorSubcoreMesh(core_axis_name="c", subcore_axis_name="s",
                                num_cores=NC, num_subcores=NS),
    scratch_types=[
        pltpu.VMEM((VC * CH,), jnp.float32),
        pltpu.VMEM((VC * CH,), jnp.float32),
        pltpu.VMEM((VC,), jnp.float32),
        pltpu.VMEM((VC,), jnp.float32),
        pltpu.VMEM((P_FLAT,), jnp.float32),
        pltpu.SemaphoreType.DMA,
        pltpu.SemaphoreType.DMA,
        pltpu.SemaphoreType.DMA,
        pltpu.SemaphoreType.DMA,
    ],
    compiler_params=pltpu.CompilerParams(use_tc_tiling_on_sc=False,
                                         needs_layout_passes=False),
)(_sc_body)

_THRESH = {0: 54.0, 1: 48.0, 2: 26.0, 3: 26.0, 4: 45.0, 5: 45.0, 6: 73.5}


def _combine(p_ref, o_ref):
  p = p_ref[...]

  def seg_sum(stat, b, ch):
    off = (stat * B * CH + b * CH + ch) * LANES
    return jnp.sum(p[:, off:off + LANES])

  def seg_max(stat, b, ch):
    off = (stat * B * CH + b * CH + ch) * LANES
    return jnp.max(p[:, off:off + LANES])

  total = jnp.float32(0.0)
  count = jnp.float32(0.0)
  for ch in range(CH):
    struct_loss = jnp.float32(0.0)
    present_any = jnp.float32(0.0)
    for b in range(B):
      cnt = seg_sum(1, b, ch)
      present = cnt > 0.0
      if ch in _MAX_CH:
        stat = seg_max(2, b, ch)
      else:
        stat = seg_sum(0, b, ch) / jnp.maximum(cnt, 1.0)
      loss_b = jnp.where(stat <= jnp.float32(_THRESH[ch]), 0.0, 1.0)
      struct_loss = struct_loss + jnp.where(present, loss_b, 0.0)
      present_any = jnp.maximum(present_any,
                                jnp.where(present, 1.0, 0.0))
    total = total + struct_loss
    count = count + present_any
  o_ref[0] = total / jnp.maximum(count, 1.0)


_combine_call = pl.pallas_call(
    _combine,
    out_shape=jax.ShapeDtypeStruct((1,), jnp.float32),
    out_specs=pl.BlockSpec(memory_space=pltpu.SMEM),
)


def kernel(predicted, structure_masks):
  pred_flat = predicted.reshape(-1)
  mask_flat = structure_masks.reshape(-1)
  partials = _sc_partials(pred_flat, mask_flat)
  partials = partials.reshape(NW, P_FLAT)
  return _combine_call(partials).astype(predicted.dtype)


# speedup vs baseline: 20.1313x; 20.1313x over previous
import functools

import jax
import jax.numpy as jnp
from jax import lax
from jax.experimental import pallas as pl
from jax.experimental.pallas import tpu as pltpu
from jax.experimental.pallas import tpu_sc as plsc

NC = 2
NS = 16
NW = NC * NS
LANES = 16

B = 2
NX = 128
PLANE = 128 * 128
NVOX = NX * PLANE
CH = 7
XPW = NX // NW
QS = 4
VC = PLANE // QS
NCHUNK = XPW * QS
TOT = B * NCHUNK
GROUPS = VC // LANES

_MAX_CH = (0, 1, 6)
_MEAN_CH = (2, 3, 4, 5)
P_ROWS = 3 * B * CH
P_FLAT = P_ROWS * LANES


def _sc_body(pred_hbm, mask_hbm, out_hbm, mb0, mb1, pb0, pb1, obuf,
             sm0, sm1, sp0, sp1):
  cid = lax.axis_index("c")
  sid = lax.axis_index("s")
  w = sid * NC + cid
  x0 = w * XPW

  mbufs = (mb0, mb1)
  pbufs = (pb0, pb1)
  msems = (sm0, sm1)
  psems = (sp0, sp1)

  neg_inf = jnp.float32(-jnp.inf)
  zero = jnp.zeros((LANES,), jnp.float32)
  one = jnp.float32(1.0)

  def start(step):
    b, rem = divmod(step, NCHUNK)
    xi, q = divmod(rem, QS)
    par = step % 2
    prow = b * NX + x0 + xi
    poff = prow * PLANE + q * VC
    moff = prow * CH * PLANE + q * VC
    copies = [
        pltpu.async_copy(mask_hbm.at[pl.ds(moff + ch * PLANE, VC)],
                         mbufs[par].at[ch], msems[par])
        for ch in range(CH)
    ]
    copies.append(pltpu.async_copy(pred_hbm.at[pl.ds(poff, VC)],
                                   pbufs[par], psems[par]))
    return copies

  pending = start(0)
  cnts = sums = maxs = None
  for step in range(TOT):
    b, k = divmod(step, NCHUNK)
    par = step % 2
    if k == 0:
      cnts = [zero] * CH
      sums = {ch: zero for ch in _MEAN_CH}
      maxs = {ch: jnp.full((LANES,), neg_inf) for ch in _MAX_CH}
    nxt = start(step + 1) if step + 1 < TOT else None
    for c in pending:
      c.wait()
    pending = nxt
    mb = mbufs[par]
    pb = pbufs[par]

    def inner(g, carry, mb=mb, pb=pb):
      cnts = list(carry[:CH])
      sums = dict(zip(_MEAN_CH, carry[CH:CH + 4]))
      maxs = dict(zip(_MAX_CH, carry[CH + 4:]))
      off = g * LANES
      pv = pb[pl.ds(off, LANES)]
      for ch in range(CH):
        mv = mb[ch, pl.ds(off, LANES)]
        m = mv > 0.0
        cnts[ch] = cnts[ch] + jnp.where(m, one, 0.0)
        if ch in _MEAN_CH:
          sums[ch] = sums[ch] + jnp.where(m, pv, 0.0)
        else:
          maxs[ch] = jnp.maximum(maxs[ch], jnp.where(m, pv, neg_inf))
      return tuple(cnts) + tuple(sums[c] for c in _MEAN_CH) + tuple(
          maxs[c] for c in _MAX_CH)

    carry = tuple(cnts) + tuple(sums[c] for c in _MEAN_CH) + tuple(
        maxs[c] for c in _MAX_CH)
    carry = lax.fori_loop(0, GROUPS, inner, carry)
    cnts = list(carry[:CH])
    sums = dict(zip(_MEAN_CH, carry[CH:CH + 4]))
    maxs = dict(zip(_MAX_CH, carry[CH + 4:]))

    if k == NCHUNK - 1:
      for ch in range(CH):
        r0 = (0 * B * CH + b * CH + ch) * LANES
        r1 = (1 * B * CH + b * CH + ch) * LANES
        r2 = (2 * B * CH + b * CH + ch) * LANES
        obuf[pl.ds(r0, LANES)] = sums[ch] if ch in _MEAN_CH else zero
        obuf[pl.ds(r1, LANES)] = cnts[ch]
        obuf[pl.ds(r2, LANES)] = (
            maxs[ch] if ch in _MAX_CH else jnp.full((LANES,), neg_inf))

  pltpu.sync_copy(obuf, out_hbm.at[pl.ds(w * P_FLAT, P_FLAT)])


_sc_partials = functools.partial(
    pl.kernel,
    out_type=jax.ShapeDtypeStruct((NW * P_FLAT,), jnp.float32),
    mesh=plsc.VectorSubcoreMesh(core_axis_name="c", subcore_axis_name="s",
                                num_cores=NC, num_subcores=NS),
    scratch_types=[
        pltpu.VMEM((CH, VC), jnp.float32),
        pltpu.VMEM((CH, VC), jnp.float32),
        pltpu.VMEM((VC,), jnp.float32),
        pltpu.VMEM((VC,), jnp.float32),
        pltpu.VMEM((P_FLAT,), jnp.float32),
        pltpu.SemaphoreType.DMA,
        pltpu.SemaphoreType.DMA,
        pltpu.SemaphoreType.DMA,
        pltpu.SemaphoreType.DMA,
    ],
    compiler_params=pltpu.CompilerParams(use_tc_tiling_on_sc=False,
                                         needs_layout_passes=False),
)(_sc_body)

_THRESH = {0: 54.0, 1: 48.0, 2: 26.0, 3: 26.0, 4: 45.0, 5: 45.0, 6: 73.5}


def _combine(p_ref, o_ref):
  p = p_ref[...]

  def seg_sum(stat, b, ch):
    off = (stat * B * CH + b * CH + ch) * LANES
    return jnp.sum(p[:, off:off + LANES])

  def seg_max(stat, b, ch):
    off = (stat * B * CH + b * CH + ch) * LANES
    return jnp.max(p[:, off:off + LANES])

  total = jnp.float32(0.0)
  count = jnp.float32(0.0)
  for ch in range(CH):
    struct_loss = jnp.float32(0.0)
    present_any = jnp.float32(0.0)
    for b in range(B):
      cnt = seg_sum(1, b, ch)
      present = cnt > 0.0
      if ch in _MAX_CH:
        stat = seg_max(2, b, ch)
      else:
        stat = seg_sum(0, b, ch) / jnp.maximum(cnt, 1.0)
      loss_b = jnp.where(stat <= jnp.float32(_THRESH[ch]), 0.0, 1.0)
      struct_loss = struct_loss + jnp.where(present, loss_b, 0.0)
      present_any = jnp.maximum(present_any,
                                jnp.where(present, 1.0, 0.0))
    total = total + struct_loss
    count = count + present_any
  o_ref[0] = total / jnp.maximum(count, 1.0)


_combine_call = pl.pallas_call(
    _combine,
    out_shape=jax.ShapeDtypeStruct((1,), jnp.float32),
    out_specs=pl.BlockSpec(memory_space=pltpu.SMEM),
)


def kernel(predicted, structure_masks):
  pred_flat = predicted.reshape(-1)
  mask_lin = structure_masks.transpose(0, 1, 4, 2, 3).reshape(-1)
  partials = _sc_partials(pred_flat, mask_lin)
  partials = partials.reshape(NW, P_FLAT)
  return _combine_call(partials).astype(predicted.dtype)
